# baseline (speedup 1.0000x reference)
import functools

import jax
import jax.numpy as jnp
from jax import lax
from jax.experimental import pallas as pl
from jax.experimental.pallas import tpu as pltpu
from jax.experimental.pallas import tpu_sc as plsc

ROWS = 16384
COLS = 200
DIM = 16
N = ROWS * COLS

NC = 2
NS = 16
NW = NC * NS
PER_W = N // NW

IDXW = 128
K = 16
CHUNK = K * IDXW
N_CHUNKS = PER_W // CHUNK


def _sc_body(ids_hbm, table_hbm, out_hbm, idx_v, rows_v, sem):
    wid = lax.axis_index("c") * NS + lax.axis_index("s")

    def chunk_body(c, carry):
        base = wid * PER_W + c * CHUNK
        pltpu.sync_copy(
            ids_hbm.at[pl.ds(pl.multiple_of(base // IDXW, 8), K)], idx_v
        )
        copies = [
            pltpu.async_copy(
                table_hbm.at[idx_v.at[j]],
                rows_v.at[pl.ds(j * IDXW, IDXW)],
                sem,
            )
            for j in range(K)
        ]
        for cp in copies:
            cp.wait()
        pltpu.sync_copy(rows_v, out_hbm.at[pl.ds(base, CHUNK)])
        return carry

    lax.fori_loop(0, N_CHUNKS, chunk_body, 0)


@jax.jit
def kernel(ids, embed):
    ids2d = ids.reshape(N // IDXW, IDXW).astype(jnp.int32)
    mesh = plsc.VectorSubcoreMesh(core_axis_name="c", subcore_axis_name="s")
    run = functools.partial(
        pl.kernel,
        out_type=jax.ShapeDtypeStruct((N, DIM), jnp.float32),
        mesh=mesh,
        scratch_types=[
            pltpu.VMEM((K, IDXW), jnp.int32),
            pltpu.VMEM((CHUNK, DIM), jnp.float32),
            pltpu.SemaphoreType.DMA,
        ],
        compiler_params=pltpu.CompilerParams(use_tc_tiling_on_sc=False),
    )(_sc_body)
    out = run(ids2d, embed)
    return out.reshape(ROWS, COLS, DIM)


# speedup vs baseline: 8.4980x; 8.4980x over previous
import functools

import jax
import jax.numpy as jnp
from jax import lax
from jax.experimental import pallas as pl
from jax.experimental.pallas import tpu as pltpu
from jax.experimental.pallas import tpu_sc as plsc

ROWS = 16384
COLS = 200
DIM = 16
N = ROWS * COLS

NC = 2
NS = 16
NW = NC * NS
PER_W = N // NW

IDXW = 128
K = 16
CHUNK = K * IDXW
N_CHUNKS = PER_W // CHUNK


def _sc_body(ids_hbm, table_hbm, out_hbm, idx_v, rows_v, tbl_v, tbl_sh, sem):
    wid = lax.axis_index("c") * NS + lax.axis_index("s")

    @pl.when(lax.axis_index("s") == 0)
    def _stage_table():
        pltpu.sync_copy(table_hbm, tbl_v)
        pltpu.sync_copy(tbl_v, tbl_sh)

    plsc.subcore_barrier()

    def chunk_body(c, carry):
        base = wid * PER_W + c * CHUNK
        pltpu.sync_copy(
            ids_hbm.at[pl.ds(pl.multiple_of(base // IDXW, 8), K)], idx_v
        )
        copies = [
            pltpu.async_copy(
                tbl_sh.at[idx_v.at[j]],
                rows_v.at[pl.ds(j * IDXW, IDXW)],
                sem,
            )
            for j in range(K)
        ]
        for cp in copies:
            cp.wait()
        pltpu.sync_copy(rows_v, out_hbm.at[pl.ds(base, CHUNK)])
        return carry

    lax.fori_loop(0, N_CHUNKS, chunk_body, 0)


@jax.jit
def kernel(ids, embed):
    ids2d = ids.reshape(N // IDXW, IDXW).astype(jnp.int32)
    mesh = plsc.VectorSubcoreMesh(core_axis_name="c", subcore_axis_name="s")
    run = functools.partial(
        pl.kernel,
        out_type=jax.ShapeDtypeStruct((N, DIM), jnp.float32),
        mesh=mesh,
        scratch_types=[
            pltpu.VMEM((K, IDXW), jnp.int32),
            pltpu.VMEM((CHUNK, DIM), jnp.float32),
            pltpu.VMEM((8, DIM), jnp.float32),
            pltpu.VMEM_SHARED((8, DIM), jnp.float32),
            pltpu.SemaphoreType.DMA,
        ],
        compiler_params=pltpu.CompilerParams(use_tc_tiling_on_sc=False),
    )(_sc_body)
    out = run(ids2d, embed)
    return out.reshape(ROWS, COLS, DIM)


# speedup vs baseline: 8.6770x; 1.0211x over previous
import functools

import jax
import jax.numpy as jnp
from jax import lax
from jax.experimental import pallas as pl
from jax.experimental.pallas import tpu as pltpu
from jax.experimental.pallas import tpu_sc as plsc

ROWS = 16384
COLS = 200
DIM = 16

NC = 2
NS = 16
NW = NC * NS
PER_W = ROWS // NW

RI = 16
N_CHUNKS = PER_W // RI


def _sc_body(ids_hbm, table_hbm, out_hbm, idx_v, rows_v, tbl_v, tbl_sh,
             isem, gsem, osem):
    wid = lax.axis_index("c") * NS + lax.axis_index("s")

    @pl.when(lax.axis_index("s") == 0)
    def _stage_table():
        pltpu.sync_copy(table_hbm, tbl_v)
        pltpu.sync_copy(tbl_v, tbl_sh)

    plsc.subcore_barrier()

    def row0(c):
        return pl.ds(pl.multiple_of(wid * PER_W + c * RI, 8), RI)

    pltpu.async_copy(ids_hbm.at[row0(0)], idx_v.at[0], isem)

    def chunk_body(c, carry):
        b = lax.rem(c, 2)

        @pl.when(c >= 2)
        def _():
            pltpu.make_async_copy(
                rows_v.at[b], out_hbm.at[pl.ds(0, RI)], osem
            ).wait()

        pltpu.make_async_copy(
            ids_hbm.at[pl.ds(0, RI)], idx_v.at[b], isem
        ).wait()

        gathers = [
            pltpu.async_copy(
                tbl_sh.at[idx_v.at[b, i]], rows_v.at[b, i], gsem
            )
            for i in range(RI)
        ]

        @pl.when(c + 1 < N_CHUNKS)
        def _():
            pltpu.async_copy(ids_hbm.at[row0(c + 1)], idx_v.at[1 - b], isem)

        for g in gathers:
            g.wait()

        pltpu.async_copy(rows_v.at[b], out_hbm.at[row0(c)], osem)
        return carry

    lax.fori_loop(0, N_CHUNKS, chunk_body, 0)

    for _ in range(2):
        pltpu.make_async_copy(
            rows_v.at[0], out_hbm.at[pl.ds(0, RI)], osem
        ).wait()


@jax.jit
def kernel(ids, embed):
    mesh = plsc.VectorSubcoreMesh(core_axis_name="c", subcore_axis_name="s")
    run = functools.partial(
        pl.kernel,
        out_type=jax.ShapeDtypeStruct((ROWS, COLS, DIM), jnp.float32),
        mesh=mesh,
        scratch_types=[
            pltpu.VMEM((2, RI, COLS), jnp.int32),
            pltpu.VMEM((2, RI, COLS, DIM), jnp.float32),
            pltpu.VMEM((8, DIM), jnp.float32),
            pltpu.VMEM_SHARED((8, DIM), jnp.float32),
            pltpu.SemaphoreType.DMA,
            pltpu.SemaphoreType.DMA,
            pltpu.SemaphoreType.DMA,
        ],
        compiler_params=pltpu.CompilerParams(use_tc_tiling_on_sc=False),
    )(_sc_body)
    return run(ids.astype(jnp.int32), embed)


# speedup vs baseline: 41.6359x; 4.7984x over previous
import functools

import jax
import jax.numpy as jnp
from jax import lax
from jax.experimental import pallas as pl
from jax.experimental.pallas import tpu as pltpu
from jax.experimental.pallas import tpu_sc as plsc

ROWS = 16384
COLS = 200
DIM = 16

NC = 2
NS = 16
NW = NC * NS
IPW = ROWS // NW

IC = 128
N_IC = IPW // IC
JC = 25
N_JC = COLS // JC
LANES = 16


def _sc_body(idst_hbm, tblt_hbm, out_hbm, ids_v, outb_v, tbl_v, osem):
    wid = lax.axis_index("c") * NS + lax.axis_index("s")

    pltpu.sync_copy(tblt_hbm, tbl_v)

    def chunk_body(ci, carry):
        i0 = pl.multiple_of((wid * N_IC + ci) * IC, IC)
        pltpu.sync_copy(idst_hbm.at[:, pl.ds(i0, IC)], ids_v)

        for jc in range(N_JC):
            b = jc % 2

            def drain():
                pltpu.make_async_copy(
                    outb_v.at[b],
                    out_hbm.at[pl.ds(0, JC), :, pl.ds(0, IC)],
                    osem,
                ).wait()

            if jc >= 2:
                drain()
            else:
                @pl.when(ci >= 1)
                def _():
                    drain()

            def col_body(jj, carry2):
                for k in range(IC // LANES):
                    ids_vec = ids_v[jc * JC + jj, pl.ds(k * LANES, LANES)]
                    for d in range(DIM):
                        got = plsc.load_gather(tbl_v, [ids_vec + d * 8])
                        outb_v[b, jj, d, pl.ds(k * LANES, LANES)] = got
                return carry2

            lax.fori_loop(0, JC, col_body, 0)

            pltpu.async_copy(
                outb_v.at[b],
                out_hbm.at[pl.ds(jc * JC, JC), :, pl.ds(i0, IC)],
                osem,
            )
        return carry

    lax.fori_loop(0, N_IC, chunk_body, 0)

    for _ in range(2):
        pltpu.make_async_copy(
            outb_v.at[0], out_hbm.at[pl.ds(0, JC), :, pl.ds(0, IC)], osem
        ).wait()


@jax.jit
def kernel(ids, embed):
    ids_t = ids.astype(jnp.int32).T
    tbl_t = embed.T.reshape(DIM * 8)
    mesh = plsc.VectorSubcoreMesh(core_axis_name="c", subcore_axis_name="s")
    run = functools.partial(
        pl.kernel,
        out_type=jax.ShapeDtypeStruct((COLS, DIM, ROWS), jnp.float32),
        mesh=mesh,
        scratch_types=[
            pltpu.VMEM((COLS, IC), jnp.int32),
            pltpu.VMEM((2, JC, DIM, IC), jnp.float32),
            pltpu.VMEM((DIM * 8,), jnp.float32),
            pltpu.SemaphoreType.DMA,
        ],
        compiler_params=pltpu.CompilerParams(
            use_tc_tiling_on_sc=True, needs_layout_passes=False
        ),
    )(_sc_body)
    out_t = run(ids_t, tbl_t)
    return out_t.transpose(2, 0, 1)


# speedup vs baseline: 97.0055x; 2.3298x over previous
import functools

import jax
import jax.numpy as jnp
from jax import lax
from jax.experimental import pallas as pl
from jax.experimental.pallas import tpu as pltpu
from jax.experimental.pallas import tpu_sc as plsc

ROWS = 16384
COLS = 200
DIM = 16

NC = 2
NS = 16
NW = NC * NS
IPW = ROWS // NW

IC = 128
N_IC = IPW // IC
JC = 25
N_JC = COLS // JC
LANES = 16


def _sc_body(idst_hbm, tblt_hbm, out_hbm, ids_v, outb_v, tbl_v, osem):
    wid = lax.axis_index("c") * NS + lax.axis_index("s")

    pltpu.sync_copy(tblt_hbm, tbl_v)

    def chunk_body(ci, carry):
        i0 = pl.multiple_of((wid * N_IC + ci) * IC, IC)
        pltpu.sync_copy(idst_hbm.at[:, pl.ds(i0, IC)], ids_v)

        for jc in range(N_JC):
            b = jc % 2

            def drain():
                pltpu.make_async_copy(
                    outb_v.at[b],
                    out_hbm.at[pl.ds(0, JC), :, pl.ds(0, IC)],
                    osem,
                ).wait()

            if jc >= 2:
                drain()
            else:
                @pl.when(ci >= 1)
                def _():
                    drain()

            def col_body(jj, carry2):
                for k in range(IC // LANES):
                    ids_vec = ids_v[jc * JC + jj, pl.ds(k * LANES, LANES)]
                    gots = [
                        plsc.load_gather(tbl_v.at[pl.ds(d * 8, 8)], [ids_vec])
                        for d in range(DIM)
                    ]
                    for d in range(DIM):
                        outb_v[b, jj, d, pl.ds(k * LANES, LANES)] = gots[d]
                return carry2

            lax.fori_loop(0, JC, col_body, 0)

            pltpu.async_copy(
                outb_v.at[b],
                out_hbm.at[pl.ds(jc * JC, JC), :, pl.ds(i0, IC)],
                osem,
            )
        return carry

    lax.fori_loop(0, N_IC, chunk_body, 0)

    for _ in range(2):
        pltpu.make_async_copy(
            outb_v.at[0], out_hbm.at[pl.ds(0, JC), :, pl.ds(0, IC)], osem
        ).wait()


@jax.jit
def kernel(ids, embed):
    ids_t = ids.astype(jnp.int32).T
    tbl_t = embed.T.reshape(DIM * 8)
    mesh = plsc.VectorSubcoreMesh(core_axis_name="c", subcore_axis_name="s")
    run = functools.partial(
        pl.kernel,
        out_type=jax.ShapeDtypeStruct((COLS, DIM, ROWS), jnp.float32),
        mesh=mesh,
        scratch_types=[
            pltpu.VMEM((COLS, IC), jnp.int32),
            pltpu.VMEM((2, JC, DIM, IC), jnp.float32),
            pltpu.VMEM((DIM * 8,), jnp.float32),
            pltpu.SemaphoreType.DMA,
        ],
        compiler_params=pltpu.CompilerParams(
            use_tc_tiling_on_sc=True, needs_layout_passes=False
        ),
    )(_sc_body)
    out_t = run(ids_t, tbl_t)
    return out_t.transpose(2, 0, 1)


# speedup vs baseline: 97.7756x; 1.0079x over previous
import functools

import jax
import jax.numpy as jnp
from jax import lax
from jax.experimental import pallas as pl
from jax.experimental.pallas import tpu as pltpu
from jax.experimental.pallas import tpu_sc as plsc

ROWS = 16384
COLS = 200
DIM = 16

NC = 2
NS = 16
NW = NC * NS
IPW = ROWS // NW

IC = 128
N_IC = IPW // IC
JC = 10
N_JC = COLS // JC
LANES = 16
NB = 3


def _sc_body(idst_hbm, tblt_hbm, out_hbm, ids_v, outb_v, tbl_v, isem, osem):
    wid = lax.axis_index("c") * NS + lax.axis_index("s")

    pltpu.sync_copy(tblt_hbm, tbl_v)

    def ids_src(ci):
        i0 = pl.multiple_of((wid * N_IC + ci) * IC, IC)
        return idst_hbm.at[:, pl.ds(i0, IC)]

    pltpu.async_copy(ids_src(0), ids_v.at[0], isem)

    def chunk_body(ci, carry):
        ib = lax.rem(ci, 2)
        i0 = pl.multiple_of((wid * N_IC + ci) * IC, IC)

        pltpu.make_async_copy(ids_src(0), ids_v.at[ib], isem).wait()

        @pl.when(ci + 1 < N_IC)
        def _():
            pltpu.async_copy(ids_src(ci + 1), ids_v.at[1 - ib], isem)

        for jc in range(N_JC):
            b = lax.rem(ci * N_JC + jc, NB)

            def drain():
                pltpu.make_async_copy(
                    outb_v.at[b],
                    out_hbm.at[pl.ds(0, JC), :, pl.ds(0, IC)],
                    osem,
                ).wait()

            if jc >= NB:
                drain()
            else:
                @pl.when(ci >= 1)
                def _():
                    drain()

            def col_body(jj, carry2):
                for k in range(IC // LANES):
                    ids_vec = ids_v[ib, jc * JC + jj, pl.ds(k * LANES, LANES)]
                    gots = [
                        plsc.load_gather(tbl_v.at[pl.ds(d * 8, 8)], [ids_vec])
                        for d in range(DIM)
                    ]
                    for d in range(DIM):
                        outb_v[b, jj, d, pl.ds(k * LANES, LANES)] = gots[d]
                return carry2

            lax.fori_loop(0, JC, col_body, 0)

            pltpu.async_copy(
                outb_v.at[b],
                out_hbm.at[pl.ds(jc * JC, JC), :, pl.ds(i0, IC)],
                osem,
            )
        return carry

    lax.fori_loop(0, N_IC, chunk_body, 0)

    for _ in range(NB):
        pltpu.make_async_copy(
            outb_v.at[0], out_hbm.at[pl.ds(0, JC), :, pl.ds(0, IC)], osem
        ).wait()


@jax.jit
def kernel(ids, embed):
    ids_t = ids.astype(jnp.int32).T
    tbl_t = embed.T.reshape(DIM * 8)
    mesh = plsc.VectorSubcoreMesh(core_axis_name="c", subcore_axis_name="s")
    run = functools.partial(
        pl.kernel,
        out_type=jax.ShapeDtypeStruct((COLS, DIM, ROWS), jnp.float32),
        mesh=mesh,
        scratch_types=[
            pltpu.VMEM((2, COLS, IC), jnp.int32),
            pltpu.VMEM((NB, JC, DIM, IC), jnp.float32),
            pltpu.VMEM((DIM * 8,), jnp.float32),
            pltpu.SemaphoreType.DMA,
            pltpu.SemaphoreType.DMA,
        ],
        compiler_params=pltpu.CompilerParams(
            use_tc_tiling_on_sc=True, needs_layout_passes=False
        ),
    )(_sc_body)
    out_t = run(ids_t, tbl_t)
    return out_t.transpose(2, 0, 1)
